# baseline (speedup 1.0000x reference)
import functools

import jax
import jax.numpy as jnp
from jax import lax
from jax.experimental import pallas as pl
from jax.experimental.pallas import tpu as pltpu
from jax.experimental.pallas import tpu_sc as plsc

F32 = jnp.float32
I32 = jnp.int32
ALPHA = 0.2
NSAMPLE = 32
NW = 32


def _fps_body(S, N, x_ref, o_ref):
    B = x_ref.shape[0]
    x = x_ref[...]
    iota = lax.broadcasted_iota(I32, (1, N), 1)

    def step(i, carry):
        dists, far = carry
        oh = iota == far
        c = jnp.sum(jnp.where(oh[:, None, :], x, 0.0), axis=2)
        o_ref[:, pl.ds(i, 1), :] = c[:, None, :]
        d = jnp.sum((x - c[:, :, None]) ** 2, axis=1)
        dists = jnp.minimum(dists, d)
        mx = jnp.max(dists, axis=1, keepdims=True)
        far = jnp.min(jnp.where(dists == mx, iota, N), axis=1, keepdims=True)
        return dists, far.astype(I32)

    lax.fori_loop(0, S, step,
                  (jnp.full((B, N), 1e10, F32), jnp.zeros((B, 1), I32)))


def _fps(xyz_t, S):
    B, _, N = xyz_t.shape
    return pl.pallas_call(
        functools.partial(_fps_body, S, N),
        out_shape=jax.ShapeDtypeStruct((B, S, 3), F32),
    )(xyz_t)


def _sc_bqg_body(B, N, S, C, r2, xyz_hbm, q_hbm, feat_hbm, out_xyz,
                 out_feat, xyz_vm, q_vm, gst, idxbuf, gidx_a, gidx_b,
                 rows_a, rows_b, sem_a, sem_b):
    SQ = B * S
    s_per = SQ // NW
    TPB = NW // B
    NCH16 = N // 16
    wid = lax.axis_index("s") * 2 + lax.axis_index("c")
    b = wid // TPB
    base_q = wid * s_per
    pltpu.sync_copy(xyz_hbm.at[b], xyz_vm)
    pltpu.sync_copy(q_hbm.at[pl.ds(base_q * 3, s_per * 3)], q_vm)
    io16 = lax.iota(I32, 16)
    zero16 = io16 * 0

    def one_query(qi, gidx):
        idxbuf[pl.ds(0, 16)] = zero16
        qx = plsc.load_gather(q_vm, [zero16 + qi * 3])
        qy = plsc.load_gather(q_vm, [zero16 + (qi * 3 + 1)])
        qz = plsc.load_gather(q_vm, [zero16 + (qi * 3 + 2)])

        UNR = 4 if N % 64 == 0 else 2
        PTS = UNR * 16

        def cond(st):
            i, cnt = st
            return jnp.logical_and(i < N // PTS, cnt < NSAMPLE)

        def body(st):
            i, cnt = st
            ms, pcs = [], []
            for u in range(UNR):
                off = i * PTS + u * 16
                xv = xyz_vm[pl.ds(off, 16)]
                yv = xyz_vm[pl.ds(N + off, 16)]
                zv = xyz_vm[pl.ds(2 * N + off, 16)]
                dx = xv - qx
                dy = yv - qy
                dz = zv - qz
                d2 = dx * dx + dy * dy + dz * dz
                m = d2 <= r2
                ms.append(m)
                pcs.append(jnp.sum(m.astype(I32)))
            off_s = cnt
            for u in range(UNR):
                plsc.store_compressed(idxbuf.at[pl.ds(off_s, 16)],
                                      i * PTS + u * 16 + io16, mask=ms[u])
                off_s = off_s + pcs[u]
            return i + 1, off_s

        _, cnt = lax.while_loop(cond, body, (jnp.zeros((), I32),
                                             jnp.zeros((), I32)))
        v0 = idxbuf[pl.ds(0, 16)]
        v1 = idxbuf[pl.ds(16, 16)]
        first = plsc.load_gather(idxbuf, [zero16])
        r0 = jnp.where(io16 < cnt, v0, first)
        r1 = jnp.where(io16 + 16 < cnt, v1, first)
        for half, rv in ((0, r0), (1, r1)):
            row = qi * NSAMPLE + half * 16 + io16
            for d in range(3):
                comp = plsc.load_gather(xyz_vm, [rv + d * N])
                plsc.store_scatter(gst, [row, zero16 + d], comp)
            if C:
                gidx[pl.ds(half * 16, 16)] = rv + b * N

    if C:
        def q2_loop(k, _):
            one_query(2 * k, gidx_a)
            cpa = pltpu.async_copy(feat_hbm.at[gidx_a], rows_a, sem_a)
            one_query(2 * k + 1, gidx_b)
            cpb = pltpu.async_copy(feat_hbm.at[gidx_b], rows_b, sem_b)
            cpa.wait()
            pltpu.sync_copy(
                rows_a, out_feat.at[pl.ds((base_q + 2 * k) * NSAMPLE,
                                          NSAMPLE), :])
            cpb.wait()
            pltpu.sync_copy(
                rows_b, out_feat.at[pl.ds((base_q + 2 * k + 1) * NSAMPLE,
                                          NSAMPLE), :])
            return 0

        lax.fori_loop(0, s_per // 2, q2_loop, 0)
    else:
        def q_loop(qi, _):
            one_query(qi, gidx_a)
            return 0

        lax.fori_loop(0, s_per, q_loop, 0)
    pltpu.sync_copy(gst, out_xyz.at[pl.ds(base_q * NSAMPLE,
                                          s_per * NSAMPLE), :])


def _sc_bq_gather(xyz_t, queries, radius, feat_tab=None):
    B, _, N = xyz_t.shape
    S = queries.shape[1]
    C = feat_tab.shape[1] if feat_tab is not None else 0
    M = B * S * NSAMPLE
    s_per = (B * S) // NW
    mesh = plsc.VectorSubcoreMesh(core_axis_name="c", subcore_axis_name="s")
    out_type = [jax.ShapeDtypeStruct((M, 4), F32)]
    if C:
        out_type.append(jax.ShapeDtypeStruct((M, C), F32))
    scratch = [
        pltpu.VMEM((3 * N,), F32),
        pltpu.VMEM((s_per * 3,), F32),
        pltpu.VMEM((s_per * NSAMPLE, 4), F32),
        pltpu.VMEM((128,), I32),
        pltpu.VMEM((NSAMPLE,), I32),
        pltpu.VMEM((NSAMPLE,), I32),
        pltpu.VMEM((NSAMPLE, max(C, 4)), F32),
        pltpu.VMEM((NSAMPLE, max(C, 4)), F32),
        pltpu.SemaphoreType.DMA,
        pltpu.SemaphoreType.DMA,
    ]
    body = functools.partial(_sc_bqg_body, B, N, S, C, radius * radius)
    cparams = pltpu.CompilerParams(needs_layout_passes=False,
                                   use_tc_tiling_on_sc=False)
    xyz_flat = xyz_t.reshape(B, 3 * N)
    q_flat = queries.reshape(-1)
    if C:
        def entry(xyz_hbm, q_hbm, feat_hbm, out_xyz, out_feat,
                  xyz_vm, q_vm, gst, idxbuf, gidx_a, gidx_b,
                  rows_a, rows_b, sem_a, sem_b):
            body(xyz_hbm, q_hbm, feat_hbm, out_xyz, out_feat,
                 xyz_vm, q_vm, gst, idxbuf, gidx_a, gidx_b,
                 rows_a, rows_b, sem_a, sem_b)
        fn = pl.kernel(entry, out_type=out_type, mesh=mesh,
                       scratch_types=scratch, compiler_params=cparams)
        return fn(xyz_flat, q_flat, feat_tab)

    def entry0(xyz_hbm, q_hbm, out_xyz, xyz_vm, q_vm, gst, idxbuf,
               gidx_a, gidx_b, rows_a, rows_b, sem_a, sem_b):
        body(xyz_hbm, q_hbm, None, out_xyz, None,
             xyz_vm, q_vm, gst, idxbuf, gidx_a, gidx_b,
             rows_a, rows_b, sem_a, sem_b)
    fn = pl.kernel(entry0, out_type=out_type, mesh=mesh,
                   scratch_types=scratch, compiler_params=cparams)
    out = fn(xyz_flat, q_flat)
    return out if isinstance(out, (list, tuple)) else (out,)


def _leaky(x):
    return jnp.where(x >= 0, x, ALPHA * x)


def _sa_mlp_body(SB, has_feat, gx_ref, gf_ref, q_ref, w1x_ref, w1f_ref,
                 b1_ref, w2_ref, b2_ref, w3_ref, b3_ref, o_ref):
    Rr = SB * NSAMPLE
    q = q_ref[0]
    qb = jnp.broadcast_to(q[:, None, :], (SB, NSAMPLE, 3)).reshape(Rr, 3)
    BF = jnp.bfloat16
    g = gx_ref[0][:, :3] - qb
    h = jnp.dot(g.astype(BF), w1x_ref[...].astype(BF),
                preferred_element_type=F32)
    if has_feat:
        h = h + jnp.dot(gf_ref[0].astype(BF), w1f_ref[...].astype(BF),
                        preferred_element_type=F32)
    h = _leaky(h + b1_ref[...])
    h = _leaky(jnp.dot(h.astype(BF), w2_ref[...].astype(BF),
                       preferred_element_type=F32) + b2_ref[...])
    h = _leaky(jnp.dot(h.astype(BF), w3_ref[...].astype(BF),
                       preferred_element_type=F32) + b3_ref[...])
    C3 = h.shape[1]
    o_ref[0] = jnp.max(h.reshape(SB, NSAMPLE, C3), axis=1)


def _sa_mlp(gx, gf, q, layers, SB):
    B, S, _ = q.shape
    (W1, b1), (W2, b2), (W3, b3) = layers
    W1x, W1f = W1[:3], W1[3:]
    has_feat = gf is not None
    if not has_feat:
        gf = jnp.zeros((B * S * NSAMPLE, 1), F32)
        W1f = jnp.zeros((1, W1.shape[1]), F32)
    C = gf.shape[1]
    C3 = W3.shape[1]
    gx3 = gx.reshape(B, S * NSAMPLE, 4)
    gf3 = gf.reshape(B, S * NSAMPLE, C)
    return pl.pallas_call(
        functools.partial(_sa_mlp_body, SB, has_feat),
        grid=(B, S // SB),
        in_specs=[
            pl.BlockSpec((1, SB * NSAMPLE, 4), lambda b, s: (b, s, 0)),
            pl.BlockSpec((1, SB * NSAMPLE, C), lambda b, s: (b, s, 0)),
            pl.BlockSpec((1, SB, 3), lambda b, s: (b, s, 0)),
            pl.BlockSpec(W1x.shape, lambda b, s: (0, 0)),
            pl.BlockSpec(W1f.shape, lambda b, s: (0, 0)),
            pl.BlockSpec((1, W1.shape[1]), lambda b, s: (0, 0)),
            pl.BlockSpec(W2.shape, lambda b, s: (0, 0)),
            pl.BlockSpec((1, W2.shape[1]), lambda b, s: (0, 0)),
            pl.BlockSpec(W3.shape, lambda b, s: (0, 0)),
            pl.BlockSpec((1, W3.shape[1]), lambda b, s: (0, 0)),
        ],
        out_specs=pl.BlockSpec((1, SB, C3), lambda b, s: (b, s, 0)),
        out_shape=jax.ShapeDtypeStruct((B, S, C3), F32),
    )(gx3, gf3, q, W1x, W1f, b1.reshape(1, -1), W2, b2.reshape(1, -1),
      W3, b3.reshape(1, -1))


def _fp_body(N2, with_head, x1_ref, x2t_ref, f1_ref, f2_ref, w1i_ref,
             w1f_ref, b1_ref, w2_ref, b2_ref, wr3_ref, wrf_ref, br_ref,
             o_ref):
    x1 = x1_ref[0]
    x2t = x2t_ref[0]
    SB = x1.shape[0]
    d2 = ((x2t[0:1, :] - x1[:, 0:1]) ** 2
          + (x2t[1:2, :] - x1[:, 1:2]) ** 2
          + (x2t[2:3, :] - x1[:, 2:3]) ** 2)
    io = lax.broadcasted_iota(I32, (1, N2), 1)
    d2m = d2
    ohs, ds_ = [], []
    for _ in range(3):
        mn = jnp.min(d2m, axis=1, keepdims=True)
        idxj = jnp.min(jnp.where(d2m == mn, io, N2), axis=1, keepdims=True)
        oh = io == idxj
        ohs.append(oh)
        ds_.append(mn)
        d2m = jnp.where(oh, jnp.float32(3e38), d2m)
    w0 = 1.0 / jnp.maximum(ds_[0], 1e-10)
    w1 = 1.0 / jnp.maximum(ds_[1], 1e-10)
    w2 = 1.0 / jnp.maximum(ds_[2], 1e-10)
    tot = w0 + w1 + w2
    wmat = (jnp.where(ohs[0], w0 / tot, 0.0)
            + jnp.where(ohs[1], w1 / tot, 0.0)
            + jnp.where(ohs[2], w2 / tot, 0.0))
    interp = jnp.dot(wmat, f2_ref[0], preferred_element_type=F32)
    h = jnp.maximum(jnp.dot(interp, w1i_ref[...], preferred_element_type=F32)
                    + jnp.dot(f1_ref[0], w1f_ref[...],
                              preferred_element_type=F32)
                    + b1_ref[...], 0.0)
    h = jnp.maximum(jnp.dot(h, w2_ref[...], preferred_element_type=F32)
                    + b2_ref[...], 0.0)
    if with_head:
        pred = (jnp.dot(x1, wr3_ref[...], preferred_element_type=F32)
                + jnp.dot(h, wrf_ref[...], preferred_element_type=F32)
                + br_ref[...])
        o_ref[0] = x1 + pred
    else:
        o_ref[0] = h


def _fp(x1, x2t, f1, f2, layers, head=None, SB=None):
    B, N1, _ = x1.shape
    N2 = x2t.shape[2]
    C1 = f1.shape[2]
    (W1, b1), (W2, b2) = layers
    C2 = W1.shape[0] - C1
    W1i, W1f = W1[:C2], W1[C2:]
    if SB is None:
        SB = N1
    if head is not None:
        Wr, br = head
        Wr3, Wrf = Wr[:3], Wr[3:]
        Cout = 3
    else:
        Wr3 = jnp.zeros((3, 1), F32)
        Wrf = jnp.zeros((W2.shape[1], 1), F32)
        br = jnp.zeros((1,), F32)
        Cout = W2.shape[1]
    return pl.pallas_call(
        functools.partial(_fp_body, N2, head is not None),
        grid=(B, N1 // SB),
        in_specs=[
            pl.BlockSpec((1, SB, 3), lambda b, s: (b, s, 0)),
            pl.BlockSpec((1, 3, N2), lambda b, s: (b, 0, 0)),
            pl.BlockSpec((1, SB, C1), lambda b, s: (b, s, 0)),
            pl.BlockSpec((1, N2, C2), lambda b, s: (b, 0, 0)),
            pl.BlockSpec(W1i.shape, lambda b, s: (0, 0)),
            pl.BlockSpec(W1f.shape, lambda b, s: (0, 0)),
            pl.BlockSpec((1, W1.shape[1]), lambda b, s: (0, 0)),
            pl.BlockSpec(W2.shape, lambda b, s: (0, 0)),
            pl.BlockSpec((1, W2.shape[1]), lambda b, s: (0, 0)),
            pl.BlockSpec(Wr3.shape, lambda b, s: (0, 0)),
            pl.BlockSpec(Wrf.shape, lambda b, s: (0, 0)),
            pl.BlockSpec((1, br.shape[0]), lambda b, s: (0, 0)),
        ],
        out_specs=pl.BlockSpec((1, SB, Cout), lambda b, s: (b, s, 0)),
        out_shape=jax.ShapeDtypeStruct((B, N1, Cout), F32),
    )(x1, x2t, f1, f2, W1i, W1f, b1.reshape(1, -1), W2, b2.reshape(1, -1),
      Wr3, Wrf, br.reshape(1, -1))


def kernel(pointcloud, hint_xyz, params):
    p = params
    xyz = pointcloud[..., :3]
    B, N0, _ = xyz.shape
    hint = hint_xyz
    S0 = hint.shape[1]

    xyz_t = jnp.transpose(xyz, (0, 2, 1))
    hint_t = jnp.transpose(hint, (0, 2, 1))

    (gx0,) = _sc_bq_gather(xyz_t, hint, 0.2)
    nx1 = _fps(hint_t, 512)
    f0 = _sa_mlp(gx0, None, hint, p['sa0'], SB=256)

    nx1_t = jnp.transpose(nx1, (0, 2, 1))
    gx1, gf1 = _sc_bq_gather(hint_t, nx1, 0.4, f0.reshape(B * S0, -1))
    f1 = _sa_mlp(gx1, gf1, nx1, p['sa1'], SB=128)

    nx2 = _fps(nx1_t, 128)
    nx2_t = jnp.transpose(nx2, (0, 2, 1))
    gx2, gf2 = _sc_bq_gather(nx1_t, nx2, 0.8, f1.reshape(B * 512, -1))
    f2 = _sa_mlp(gx2, gf2, nx2, p['sa2'], SB=64)

    o1 = _fp(nx1, nx2_t, f1, f2, p['fp1'])

    out = _fp(hint, nx1_t, f0, o1, p['fp0'], head=p['reg'], SB=512)
    return out


# speedup vs baseline: 1.0051x; 1.0051x over previous
import functools

import jax
import jax.numpy as jnp
from jax import lax
from jax.experimental import pallas as pl
from jax.experimental.pallas import tpu as pltpu
from jax.experimental.pallas import tpu_sc as plsc

F32 = jnp.float32
I32 = jnp.int32
ALPHA = 0.2
NSAMPLE = 32
NW = 32


def _fps_body(S, N, x_ref, o_ref):
    B = x_ref.shape[0]
    x = x_ref[...]
    iota = lax.broadcasted_iota(I32, (1, N), 1)

    def step(i, carry):
        dists, far = carry
        oh = iota == far
        c = jnp.sum(jnp.where(oh[:, None, :], x, 0.0), axis=2)
        o_ref[:, pl.ds(i, 1), :] = c[:, None, :]
        d = jnp.sum((x - c[:, :, None]) ** 2, axis=1)
        dists = jnp.minimum(dists, d)
        mx = jnp.max(dists, axis=1, keepdims=True)
        far = jnp.min(jnp.where(dists == mx, iota, N), axis=1, keepdims=True)
        return dists, far.astype(I32)

    lax.fori_loop(0, S, step,
                  (jnp.full((B, N), 1e10, F32), jnp.zeros((B, 1), I32)))


def _fps(xyz_t, S):
    B, _, N = xyz_t.shape
    return pl.pallas_call(
        functools.partial(_fps_body, S, N),
        out_shape=jax.ShapeDtypeStruct((B, S, 3), F32),
    )(xyz_t)


def _sc_bqg_body(B, N, S, C, r2, xyz_hbm, q_hbm, feat_hbm, out_xyz,
                 out_feat, xyz_vm, q_vm, gst, idxbuf, gidx_a, gidx_b,
                 rows_a, rows_b, sem_a, sem_b):
    SQ = B * S
    s_per = SQ // NW
    TPB = NW // B
    NCH16 = N // 16
    wid = lax.axis_index("s") * 2 + lax.axis_index("c")
    b = wid // TPB
    base_q = wid * s_per
    pltpu.sync_copy(xyz_hbm.at[b], xyz_vm)
    pltpu.sync_copy(q_hbm.at[pl.ds(base_q * 3, s_per * 3)], q_vm)
    io16 = lax.iota(I32, 16)
    zero16 = io16 * 0

    def one_query(qi, gidx):
        idxbuf[pl.ds(0, 16)] = zero16
        qx = plsc.load_gather(q_vm, [zero16 + qi * 3])
        qy = plsc.load_gather(q_vm, [zero16 + (qi * 3 + 1)])
        qz = plsc.load_gather(q_vm, [zero16 + (qi * 3 + 2)])

        UNR = 4 if N % 64 == 0 else 2
        PTS = UNR * 16

        def cond(st):
            i, cnt = st
            return jnp.logical_and(i < N // PTS, cnt < NSAMPLE)

        def body(st):
            i, cnt = st
            ms, pcs = [], []
            for u in range(UNR):
                off = i * PTS + u * 16
                xv = xyz_vm[pl.ds(off, 16)]
                yv = xyz_vm[pl.ds(N + off, 16)]
                zv = xyz_vm[pl.ds(2 * N + off, 16)]
                dx = xv - qx
                dy = yv - qy
                dz = zv - qz
                d2 = dx * dx + dy * dy + dz * dz
                m = d2 <= r2
                ms.append(m)
                pcs.append(jnp.sum(m.astype(I32)))
            off_s = cnt
            for u in range(UNR):
                plsc.store_compressed(idxbuf.at[pl.ds(off_s, 16)],
                                      i * PTS + u * 16 + io16, mask=ms[u])
                off_s = off_s + pcs[u]
            return i + 1, off_s

        _, cnt = lax.while_loop(cond, body, (jnp.zeros((), I32),
                                             jnp.zeros((), I32)))
        v0 = idxbuf[pl.ds(0, 16)]
        v1 = idxbuf[pl.ds(16, 16)]
        first = plsc.load_gather(idxbuf, [zero16])
        r0 = jnp.where(io16 < cnt, v0, first)
        r1 = jnp.where(io16 + 16 < cnt, v1, first)
        for half, rv in ((0, r0), (1, r1)):
            row = qi * NSAMPLE + half * 16 + io16
            for d in range(3):
                comp = plsc.load_gather(xyz_vm, [rv + d * N])
                plsc.store_scatter(gst, [row, zero16 + d], comp)
            if C:
                gidx[pl.ds(half * 16, 16)] = rv + b * N

    if C:
        def q2_loop(k, _):
            one_query(2 * k, gidx_a)
            cpa = pltpu.async_copy(feat_hbm.at[gidx_a], rows_a, sem_a)
            one_query(2 * k + 1, gidx_b)
            cpb = pltpu.async_copy(feat_hbm.at[gidx_b], rows_b, sem_b)
            cpa.wait()
            pltpu.sync_copy(
                rows_a, out_feat.at[pl.ds((base_q + 2 * k) * NSAMPLE,
                                          NSAMPLE), :])
            cpb.wait()
            pltpu.sync_copy(
                rows_b, out_feat.at[pl.ds((base_q + 2 * k + 1) * NSAMPLE,
                                          NSAMPLE), :])
            return 0

        lax.fori_loop(0, s_per // 2, q2_loop, 0)
    else:
        def q_loop(qi, _):
            one_query(qi, gidx_a)
            return 0

        lax.fori_loop(0, s_per, q_loop, 0)
    pltpu.sync_copy(gst, out_xyz.at[pl.ds(base_q * NSAMPLE,
                                          s_per * NSAMPLE), :])


def _sc_bq_gather(xyz_t, queries, radius, feat_tab=None):
    B, _, N = xyz_t.shape
    S = queries.shape[1]
    C = feat_tab.shape[1] if feat_tab is not None else 0
    M = B * S * NSAMPLE
    s_per = (B * S) // NW
    mesh = plsc.VectorSubcoreMesh(core_axis_name="c", subcore_axis_name="s")
    out_type = [jax.ShapeDtypeStruct((M, 4), F32)]
    if C:
        out_type.append(jax.ShapeDtypeStruct((M, C), F32))
    scratch = [
        pltpu.VMEM((3 * N,), F32),
        pltpu.VMEM((s_per * 3,), F32),
        pltpu.VMEM((s_per * NSAMPLE, 4), F32),
        pltpu.VMEM((128,), I32),
        pltpu.VMEM((NSAMPLE,), I32),
        pltpu.VMEM((NSAMPLE,), I32),
        pltpu.VMEM((NSAMPLE, max(C, 4)), F32),
        pltpu.VMEM((NSAMPLE, max(C, 4)), F32),
        pltpu.SemaphoreType.DMA,
        pltpu.SemaphoreType.DMA,
    ]
    body = functools.partial(_sc_bqg_body, B, N, S, C, radius * radius)
    cparams = pltpu.CompilerParams(needs_layout_passes=False,
                                   use_tc_tiling_on_sc=False)
    xyz_flat = xyz_t.reshape(B, 3 * N)
    q_flat = queries.reshape(-1)
    if C:
        def entry(xyz_hbm, q_hbm, feat_hbm, out_xyz, out_feat,
                  xyz_vm, q_vm, gst, idxbuf, gidx_a, gidx_b,
                  rows_a, rows_b, sem_a, sem_b):
            body(xyz_hbm, q_hbm, feat_hbm, out_xyz, out_feat,
                 xyz_vm, q_vm, gst, idxbuf, gidx_a, gidx_b,
                 rows_a, rows_b, sem_a, sem_b)
        fn = pl.kernel(entry, out_type=out_type, mesh=mesh,
                       scratch_types=scratch, compiler_params=cparams)
        return fn(xyz_flat, q_flat, feat_tab)

    def entry0(xyz_hbm, q_hbm, out_xyz, xyz_vm, q_vm, gst, idxbuf,
               gidx_a, gidx_b, rows_a, rows_b, sem_a, sem_b):
        body(xyz_hbm, q_hbm, None, out_xyz, None,
             xyz_vm, q_vm, gst, idxbuf, gidx_a, gidx_b,
             rows_a, rows_b, sem_a, sem_b)
    fn = pl.kernel(entry0, out_type=out_type, mesh=mesh,
                   scratch_types=scratch, compiler_params=cparams)
    out = fn(xyz_flat, q_flat)
    return out if isinstance(out, (list, tuple)) else (out,)


def _leaky(x):
    return jnp.where(x >= 0, x, ALPHA * x)


def _sa_mlp_body(SB, has_feat, gx_ref, gf_ref, q_ref, w1x_ref, w1f_ref,
                 b1_ref, w2_ref, b2_ref, w3_ref, b3_ref, o_ref):
    Rr = SB * NSAMPLE
    q = q_ref[0]
    qb = jnp.broadcast_to(q[:, None, :], (SB, NSAMPLE, 3)).reshape(Rr, 3)
    g = gx_ref[0][:, :3] - qb
    h = jnp.dot(g, w1x_ref[...], preferred_element_type=F32)
    if has_feat:
        h = h + jnp.dot(gf_ref[0], w1f_ref[...], preferred_element_type=F32)
    h = _leaky(h + b1_ref[...])
    h = _leaky(jnp.dot(h, w2_ref[...], preferred_element_type=F32)
               + b2_ref[...])
    h = _leaky(jnp.dot(h, w3_ref[...], preferred_element_type=F32)
               + b3_ref[...])
    C3 = h.shape[1]
    o_ref[0] = jnp.max(h.reshape(SB, NSAMPLE, C3), axis=1)


def _sa_mlp(gx, gf, q, layers, SB):
    B, S, _ = q.shape
    (W1, b1), (W2, b2), (W3, b3) = layers
    W1x, W1f = W1[:3], W1[3:]
    has_feat = gf is not None
    if not has_feat:
        gf = jnp.zeros((B * S * NSAMPLE, 1), F32)
        W1f = jnp.zeros((1, W1.shape[1]), F32)
    C = gf.shape[1]
    C3 = W3.shape[1]
    gx3 = gx.reshape(B, S * NSAMPLE, 4)
    gf3 = gf.reshape(B, S * NSAMPLE, C)
    return pl.pallas_call(
        functools.partial(_sa_mlp_body, SB, has_feat),
        grid=(B, S // SB),
        in_specs=[
            pl.BlockSpec((1, SB * NSAMPLE, 4), lambda b, s: (b, s, 0)),
            pl.BlockSpec((1, SB * NSAMPLE, C), lambda b, s: (b, s, 0)),
            pl.BlockSpec((1, SB, 3), lambda b, s: (b, s, 0)),
            pl.BlockSpec(W1x.shape, lambda b, s: (0, 0)),
            pl.BlockSpec(W1f.shape, lambda b, s: (0, 0)),
            pl.BlockSpec((1, W1.shape[1]), lambda b, s: (0, 0)),
            pl.BlockSpec(W2.shape, lambda b, s: (0, 0)),
            pl.BlockSpec((1, W2.shape[1]), lambda b, s: (0, 0)),
            pl.BlockSpec(W3.shape, lambda b, s: (0, 0)),
            pl.BlockSpec((1, W3.shape[1]), lambda b, s: (0, 0)),
        ],
        out_specs=pl.BlockSpec((1, SB, C3), lambda b, s: (b, s, 0)),
        out_shape=jax.ShapeDtypeStruct((B, S, C3), F32),
    )(gx3, gf3, q, W1x, W1f, b1.reshape(1, -1), W2, b2.reshape(1, -1),
      W3, b3.reshape(1, -1))


def _fp_body(N2, with_head, x1_ref, x2t_ref, f1_ref, f2_ref, w1i_ref,
             w1f_ref, b1_ref, w2_ref, b2_ref, wr3_ref, wrf_ref, br_ref,
             o_ref):
    x1 = x1_ref[0]
    x2t = x2t_ref[0]
    SB = x1.shape[0]
    d2 = ((x2t[0:1, :] - x1[:, 0:1]) ** 2
          + (x2t[1:2, :] - x1[:, 1:2]) ** 2
          + (x2t[2:3, :] - x1[:, 2:3]) ** 2)
    io = lax.broadcasted_iota(I32, (1, N2), 1)
    d2m = d2
    ohs, ds_ = [], []
    for _ in range(3):
        mn = jnp.min(d2m, axis=1, keepdims=True)
        idxj = jnp.min(jnp.where(d2m == mn, io, N2), axis=1, keepdims=True)
        oh = io == idxj
        ohs.append(oh)
        ds_.append(mn)
        d2m = jnp.where(oh, jnp.float32(3e38), d2m)
    w0 = 1.0 / jnp.maximum(ds_[0], 1e-10)
    w1 = 1.0 / jnp.maximum(ds_[1], 1e-10)
    w2 = 1.0 / jnp.maximum(ds_[2], 1e-10)
    tot = w0 + w1 + w2
    wmat = (jnp.where(ohs[0], w0 / tot, 0.0)
            + jnp.where(ohs[1], w1 / tot, 0.0)
            + jnp.where(ohs[2], w2 / tot, 0.0))
    interp = jnp.dot(wmat, f2_ref[0], preferred_element_type=F32)
    h = jnp.maximum(jnp.dot(interp, w1i_ref[...], preferred_element_type=F32)
                    + jnp.dot(f1_ref[0], w1f_ref[...],
                              preferred_element_type=F32)
                    + b1_ref[...], 0.0)
    h = jnp.maximum(jnp.dot(h, w2_ref[...], preferred_element_type=F32)
                    + b2_ref[...], 0.0)
    if with_head:
        pred = (jnp.dot(x1, wr3_ref[...], preferred_element_type=F32)
                + jnp.dot(h, wrf_ref[...], preferred_element_type=F32)
                + br_ref[...])
        o_ref[0] = x1 + pred
    else:
        o_ref[0] = h


def _fp(x1, x2t, f1, f2, layers, head=None, SB=None):
    B, N1, _ = x1.shape
    N2 = x2t.shape[2]
    C1 = f1.shape[2]
    (W1, b1), (W2, b2) = layers
    C2 = W1.shape[0] - C1
    W1i, W1f = W1[:C2], W1[C2:]
    if SB is None:
        SB = N1
    if head is not None:
        Wr, br = head
        Wr3, Wrf = Wr[:3], Wr[3:]
        Cout = 3
    else:
        Wr3 = jnp.zeros((3, 1), F32)
        Wrf = jnp.zeros((W2.shape[1], 1), F32)
        br = jnp.zeros((1,), F32)
        Cout = W2.shape[1]
    return pl.pallas_call(
        functools.partial(_fp_body, N2, head is not None),
        grid=(B, N1 // SB),
        in_specs=[
            pl.BlockSpec((1, SB, 3), lambda b, s: (b, s, 0)),
            pl.BlockSpec((1, 3, N2), lambda b, s: (b, 0, 0)),
            pl.BlockSpec((1, SB, C1), lambda b, s: (b, s, 0)),
            pl.BlockSpec((1, N2, C2), lambda b, s: (b, 0, 0)),
            pl.BlockSpec(W1i.shape, lambda b, s: (0, 0)),
            pl.BlockSpec(W1f.shape, lambda b, s: (0, 0)),
            pl.BlockSpec((1, W1.shape[1]), lambda b, s: (0, 0)),
            pl.BlockSpec(W2.shape, lambda b, s: (0, 0)),
            pl.BlockSpec((1, W2.shape[1]), lambda b, s: (0, 0)),
            pl.BlockSpec(Wr3.shape, lambda b, s: (0, 0)),
            pl.BlockSpec(Wrf.shape, lambda b, s: (0, 0)),
            pl.BlockSpec((1, br.shape[0]), lambda b, s: (0, 0)),
        ],
        out_specs=pl.BlockSpec((1, SB, Cout), lambda b, s: (b, s, 0)),
        out_shape=jax.ShapeDtypeStruct((B, N1, Cout), F32),
    )(x1, x2t, f1, f2, W1i, W1f, b1.reshape(1, -1), W2, b2.reshape(1, -1),
      Wr3, Wrf, br.reshape(1, -1))


def kernel(pointcloud, hint_xyz, params):
    p = params
    xyz = pointcloud[..., :3]
    B, N0, _ = xyz.shape
    hint = hint_xyz
    S0 = hint.shape[1]

    xyz_t = jnp.transpose(xyz, (0, 2, 1))
    hint_t = jnp.transpose(hint, (0, 2, 1))

    (gx0,) = _sc_bq_gather(xyz_t, hint, 0.2)
    nx1 = _fps(hint_t, 512)
    f0 = _sa_mlp(gx0, None, hint, p['sa0'], SB=256)

    nx1_t = jnp.transpose(nx1, (0, 2, 1))
    gx1, gf1 = _sc_bq_gather(hint_t, nx1, 0.4, f0.reshape(B * S0, -1))
    f1 = _sa_mlp(gx1, gf1, nx1, p['sa1'], SB=128)

    nx2 = _fps(nx1_t, 128)
    nx2_t = jnp.transpose(nx2, (0, 2, 1))
    gx2, gf2 = _sc_bq_gather(nx1_t, nx2, 0.8, f1.reshape(B * 512, -1))
    f2 = _sa_mlp(gx2, gf2, nx2, p['sa2'], SB=64)

    o1 = _fp(nx1, nx2_t, f1, f2, p['fp1'])

    out = _fp(hint, nx1_t, f0, o1, p['fp0'], head=p['reg'], SB=512)
    return out
